# baseline (speedup 1.0000x reference)
import functools

import jax
import jax.numpy as jnp
import numpy as np
from jax import lax
from jax.experimental import pallas as pl
from jax.experimental.pallas import tpu as pltpu
from jax.experimental.pallas import tpu_sc as plsc

K = 12
NC, NS, L = 2, 16, 16
NW = NC * NS

_BP = np.linspace(0.0, 1.0, K + 1, endpoint=False)[1:].astype(np.float32)


def _sc_trend(t_flat, params, *, n):
    ch = n // NW
    nb = 4
    ck = ch // nb
    mesh = plsc.VectorSubcoreMesh(core_axis_name="c", subcore_axis_name="s")

    @functools.partial(
        pl.kernel,
        mesh=mesh,
        out_type=jax.ShapeDtypeStruct((n,), jnp.float32),
        compiler_params=pltpu.CompilerParams(needs_layout_passes=False),
        scratch_types=[
            pltpu.VMEM((3 * L,), jnp.float32),
            pltpu.VMEM((L,), jnp.float32),
            pltpu.VMEM((L,), jnp.float32),
            [pltpu.VMEM((ck,), jnp.float32)] * 4,
            [pltpu.VMEM((ck,), jnp.float32)] * 4,
            [pltpu.SemaphoreType.DMA] * 4,
            [pltpu.SemaphoreType.DMA] * 4,
        ],
    )
    def body(t_hbm, params_hbm, out_hbm, params_v, wtab_v, btab_v,
             t_v, o_v, in_sem, out_sem):
        wid = lax.axis_index("s") * NC + lax.axis_index("c")
        base = wid * ch

        pltpu.sync_copy(params_hbm, params_v)
        lanes = lax.iota(jnp.int32, L)
        wt = params_v[pl.ds(0, L)]
        bt = params_v[pl.ds(L, L)]
        for i in range(K):
            di = plsc.load_gather(
                params_v, [jnp.full((L,), 2 * L + i, jnp.int32)])
            m = lanes >= (i + 1)
            wt = wt + jnp.where(m, di, np.float32(0.0))
            bt = bt - jnp.where(m, di * _BP[i], np.float32(0.0))
        wtab_v[...] = wt
        btab_v[...] = bt

        in_h = [pltpu.async_copy(
            t_hbm.at[pl.ds(base + c * ck, ck)], t_v[c], in_sem[c])
            for c in range(nb)]
        out_h = [None] * nb
        for c in range(nb):
            in_h[c].wait()

            @plsc.parallel_loop(0, ck, step=L, unroll=8)
            def _loop(i, tin=t_v[c], tout=o_v[c]):
                tv = tin[pl.ds(i, L)]
                idx = (tv * np.float32(np.nextafter(K + 1, 0, dtype=np.float32))).astype(jnp.int32)
                wv = plsc.load_gather(wtab_v, [idx])
                bv = plsc.load_gather(btab_v, [idx])
                tout[pl.ds(i, L)] = tv * wv + bv

            out_h[c] = pltpu.async_copy(
                o_v[c], out_hbm.at[pl.ds(base + c * ck, ck)], out_sem[c])
        for c in range(nb):
            out_h[c].wait()

    return body(t_flat, params)


def kernel(t, weight, bias, deltas):
    n = t.shape[0]
    t_flat = t.reshape(n).astype(jnp.float32)
    w16 = jnp.full((L,), weight[0, 0], jnp.float32)
    b16 = jnp.full((L,), bias[0], jnp.float32)
    d16 = jnp.concatenate(
        [deltas.astype(jnp.float32), jnp.zeros((L - K,), jnp.float32)])
    params = jnp.concatenate([w16, b16, d16])
    out = _sc_trend(t_flat, params, n=n)
    return out.reshape(n, 1)


# speedup vs baseline: 1.0361x; 1.0361x over previous
import functools

import jax
import jax.numpy as jnp
import numpy as np
from jax import lax
from jax.experimental import pallas as pl
from jax.experimental.pallas import tpu as pltpu
from jax.experimental.pallas import tpu_sc as plsc

K = 12
NC, NS, L = 2, 16, 16
NW = NC * NS

_BP = np.linspace(0.0, 1.0, K + 1, endpoint=False)[1:].astype(np.float32)
_SCALE = np.float32(np.nextafter(K + 1, 0, dtype=np.float32))


def _sc_trend(t_flat, params, *, n):
    ch = n // NW
    nb = 4
    ck = ch // nb
    mesh = plsc.VectorSubcoreMesh(core_axis_name="c", subcore_axis_name="s")

    @functools.partial(
        pl.kernel,
        mesh=mesh,
        out_type=jax.ShapeDtypeStruct((n,), jnp.float32),
        compiler_params=pltpu.CompilerParams(needs_layout_passes=False, use_tc_tiling_on_sc=False),
        scratch_types=[
            pltpu.VMEM((3 * L,), jnp.float32),
            pltpu.VMEM((L,), jnp.float32),
            pltpu.VMEM((L,), jnp.float32),
            [pltpu.VMEM((ck,), jnp.float32)] * 2,
            [pltpu.VMEM((ck,), jnp.float32)] * 2,
            [pltpu.SemaphoreType.DMA] * 2,
            [pltpu.SemaphoreType.DMA] * 2,
        ],
    )
    def body(t_hbm, params_hbm, out_hbm, params_v, wtab_v, btab_v,
             t_v, o_v, in_sem, out_sem):
        wid = lax.axis_index("s") * NC + lax.axis_index("c")
        base = wid * ch

        pltpu.sync_copy(params_hbm, params_v)
        lanes = lax.iota(jnp.int32, L)
        wt = params_v[pl.ds(0, L)]
        bt = params_v[pl.ds(L, L)]
        for i in range(K):
            di = plsc.load_gather(
                params_v, [jnp.full((L,), 2 * L + i, jnp.int32)])
            m = lanes >= (i + 1)
            wt = wt + jnp.where(m, di, np.float32(0.0))
            bt = bt - jnp.where(m, di * _BP[i], np.float32(0.0))
        wtab_v[...] = wt * np.float32(1.0 / _SCALE)
        btab_v[...] = bt

        in_h = [None, None]
        out_h = [None, None]
        in_h[0] = pltpu.async_copy(
            t_hbm.at[pl.ds(base, ck)], t_v[0], in_sem[0])
        for c in range(nb):
            s = c % 2
            if c + 1 < nb:
                in_h[(c + 1) % 2] = pltpu.async_copy(
                    t_hbm.at[pl.ds(base + (c + 1) * ck, ck)],
                    t_v[(c + 1) % 2], in_sem[(c + 1) % 2])
            in_h[s].wait()
            if c >= 2:
                out_h[s].wait()

            @plsc.parallel_loop(0, ck, step=L, unroll=8)
            def _loop(i, tin=t_v[s], tout=o_v[s]):
                ts = tin[pl.ds(i, L)] * _SCALE
                idx = ts.astype(jnp.int32)
                wv = plsc.load_gather(wtab_v, [idx])
                bv = plsc.load_gather(btab_v, [idx])
                tout[pl.ds(i, L)] = ts * wv + bv

            out_h[s] = pltpu.async_copy(
                o_v[s], out_hbm.at[pl.ds(base + c * ck, ck)], out_sem[s])
        out_h[(nb - 2) % 2].wait()
        out_h[(nb - 1) % 2].wait()

    return body(t_flat, params)


def kernel(t, weight, bias, deltas):
    n = t.shape[0]
    t_flat = t.reshape(n).astype(jnp.float32)
    w16 = jnp.full((L,), weight[0, 0], jnp.float32)
    b16 = jnp.full((L,), bias[0], jnp.float32)
    d16 = jnp.concatenate(
        [deltas.astype(jnp.float32), jnp.zeros((L - K,), jnp.float32)])
    params = jnp.concatenate([w16, b16, d16])
    out = _sc_trend(t_flat, params, n=n)
    return out.reshape(n, 1)


# speedup vs baseline: 1.0388x; 1.0026x over previous
import functools

import jax
import jax.numpy as jnp
import numpy as np
from jax import lax
from jax.experimental import pallas as pl
from jax.experimental.pallas import tpu as pltpu
from jax.experimental.pallas import tpu_sc as plsc

K = 12
NC, NS, L = 2, 16, 16
NW = NC * NS

_BP = np.linspace(0.0, 1.0, K + 1, endpoint=False)[1:].astype(np.float32)
_SCALE = np.float32(np.nextafter(K + 1, 0, dtype=np.float32))


def _sc_trend(t_flat, params, *, n):
    ch = n // NW
    nb = 2
    ck = ch // nb
    mesh = plsc.VectorSubcoreMesh(core_axis_name="c", subcore_axis_name="s")

    @functools.partial(
        pl.kernel,
        mesh=mesh,
        out_type=jax.ShapeDtypeStruct((n,), jnp.float32),
        compiler_params=pltpu.CompilerParams(needs_layout_passes=False, use_tc_tiling_on_sc=False),
        scratch_types=[
            pltpu.VMEM((3 * L,), jnp.float32),
            pltpu.VMEM((L,), jnp.float32),
            pltpu.VMEM((L,), jnp.float32),
            [pltpu.VMEM((ck,), jnp.float32)] * 2,
            [pltpu.VMEM((ck,), jnp.float32)] * 2,
            [pltpu.SemaphoreType.DMA] * 2,
            [pltpu.SemaphoreType.DMA] * 2,
        ],
    )
    def body(t_hbm, params_hbm, out_hbm, params_v, wtab_v, btab_v,
             t_v, o_v, in_sem, out_sem):
        wid = lax.axis_index("s") * NC + lax.axis_index("c")
        base = wid * ch

        pltpu.sync_copy(params_hbm, params_v)
        lanes = lax.iota(jnp.int32, L)
        wt = params_v[pl.ds(0, L)]
        bt = params_v[pl.ds(L, L)]
        for i in range(K):
            di = plsc.load_gather(
                params_v, [jnp.full((L,), 2 * L + i, jnp.int32)])
            m = lanes >= (i + 1)
            wt = wt + jnp.where(m, di, np.float32(0.0))
            bt = bt - jnp.where(m, di * _BP[i], np.float32(0.0))
        wtab_v[...] = wt * np.float32(1.0 / _SCALE)
        btab_v[...] = bt

        in_h = [None, None]
        out_h = [None, None]
        in_h[0] = pltpu.async_copy(
            t_hbm.at[pl.ds(base, ck)], t_v[0], in_sem[0])
        for c in range(nb):
            s = c % 2
            if c + 1 < nb:
                in_h[(c + 1) % 2] = pltpu.async_copy(
                    t_hbm.at[pl.ds(base + (c + 1) * ck, ck)],
                    t_v[(c + 1) % 2], in_sem[(c + 1) % 2])
            in_h[s].wait()
            if c >= 2:
                out_h[s].wait()

            @plsc.parallel_loop(0, ck, step=L, unroll=8)
            def _loop(i, tin=t_v[s], tout=o_v[s]):
                ts = tin[pl.ds(i, L)] * _SCALE
                idx = ts.astype(jnp.int32)
                wv = plsc.load_gather(wtab_v, [idx])
                bv = plsc.load_gather(btab_v, [idx])
                tout[pl.ds(i, L)] = ts * wv + bv

            out_h[s] = pltpu.async_copy(
                o_v[s], out_hbm.at[pl.ds(base + c * ck, ck)], out_sem[s])
        out_h[(nb - 2) % 2].wait()
        out_h[(nb - 1) % 2].wait()

    return body(t_flat, params)


def kernel(t, weight, bias, deltas):
    n = t.shape[0]
    t_flat = t.reshape(n).astype(jnp.float32)
    w16 = jnp.full((L,), weight[0, 0], jnp.float32)
    b16 = jnp.full((L,), bias[0], jnp.float32)
    d16 = jnp.concatenate(
        [deltas.astype(jnp.float32), jnp.zeros((L - K,), jnp.float32)])
    params = jnp.concatenate([w16, b16, d16])
    out = _sc_trend(t_flat, params, n=n)
    return out.reshape(n, 1)


# speedup vs baseline: 1.0395x; 1.0007x over previous
import functools

import jax
import jax.numpy as jnp
import numpy as np
from jax import lax
from jax.experimental import pallas as pl
from jax.experimental.pallas import tpu as pltpu
from jax.experimental.pallas import tpu_sc as plsc

K = 12
NC, NS, L = 2, 16, 16
NW = NC * NS

_BP = np.linspace(0.0, 1.0, K + 1, endpoint=False)[1:].astype(np.float32)
_SCALE = np.float32(np.nextafter(K + 1, 0, dtype=np.float32))


def _sc_trend(t_flat, params, *, n):
    ch = n // NW
    nb = 2
    ck = ch // nb
    mesh = plsc.VectorSubcoreMesh(core_axis_name="c", subcore_axis_name="s")

    @functools.partial(
        pl.kernel,
        mesh=mesh,
        out_type=jax.ShapeDtypeStruct((n,), jnp.float32),
        compiler_params=pltpu.CompilerParams(needs_layout_passes=False, use_tc_tiling_on_sc=False),
        scratch_types=[
            pltpu.VMEM((3 * L,), jnp.float32),
            pltpu.VMEM((L,), jnp.float32),
            pltpu.VMEM((L,), jnp.float32),
            [pltpu.VMEM((ck,), jnp.float32)] * 2,
            [pltpu.VMEM((ck,), jnp.float32)] * 2,
            [pltpu.SemaphoreType.DMA] * 2,
            [pltpu.SemaphoreType.DMA] * 2,
        ],
    )
    def body(t_hbm, params_hbm, out_hbm, params_v, wtab_v, btab_v,
             t_v, o_v, in_sem, out_sem):
        wid = lax.axis_index("s") * NC + lax.axis_index("c")
        base = wid * ch

        pltpu.sync_copy(params_hbm, params_v)
        lanes = lax.iota(jnp.int32, L)
        wt = params_v[pl.ds(0, L)]
        bt = params_v[pl.ds(L, L)]
        for i in range(K):
            di = plsc.load_gather(
                params_v, [jnp.full((L,), 2 * L + i, jnp.int32)])
            m = lanes >= (i + 1)
            wt = wt + jnp.where(m, di, np.float32(0.0))
            bt = bt - jnp.where(m, di * _BP[i], np.float32(0.0))
        wtab_v[...] = wt * np.float32(1.0 / _SCALE)
        btab_v[...] = bt

        in_h = [None, None]
        out_h = [None, None]
        in_h[0] = pltpu.async_copy(
            t_hbm.at[pl.ds(base, ck)], t_v[0], in_sem[0])
        for c in range(nb):
            s = c % 2
            if c + 1 < nb:
                in_h[(c + 1) % 2] = pltpu.async_copy(
                    t_hbm.at[pl.ds(base + (c + 1) * ck, ck)],
                    t_v[(c + 1) % 2], in_sem[(c + 1) % 2])
            in_h[s].wait()
            if c >= 2:
                out_h[s].wait()

            @plsc.parallel_loop(0, ck, step=L, unroll=8)
            def _loop(i, tin=t_v[s], tout=o_v[s]):
                ts = tin[pl.ds(i, L)] * _SCALE
                idx = ts.astype(jnp.int32)
                wv = plsc.load_gather(wtab_v, [idx])
                bv = plsc.load_gather(btab_v, [idx])
                tout[pl.ds(i, L)] = ts * wv + bv

            out_h[s] = pltpu.async_copy(
                o_v[s], out_hbm.at[pl.ds(base + c * ck, ck)], out_sem[s])
        for c in range(max(0, nb - 2), nb):
            out_h[c % 2].wait()

    return body(t_flat, params)


def kernel(t, weight, bias, deltas):
    n = t.shape[0]
    t_flat = t.reshape(n).astype(jnp.float32)
    w16 = jnp.full((L,), weight[0, 0], jnp.float32)
    b16 = jnp.full((L,), bias[0], jnp.float32)
    d16 = jnp.concatenate(
        [deltas.astype(jnp.float32), jnp.zeros((L - K,), jnp.float32)])
    params = jnp.concatenate([w16, b16, d16])
    out = _sc_trend(t_flat, params, n=n)
    return out.reshape(n, 1)
